# baseline (speedup 1.0000x reference)
import functools
import jax
import jax.numpy as jnp
from jax import lax
from jax.experimental import pallas as pl
from jax.experimental.pallas import tpu as pltpu
from jax.experimental.pallas import tpu_sc as plsc

N = 10000
E = 320000
D = 128
NG = D // 16

NC = 2
NS = 16
NW = NC * NS
EPT = E // NW
CH = 128
NFULL = EPT // CH
TAIL = EPT - NFULL * CH
NP = 10240
ROWS_PT = NP // NS


def _gate_rows(rows_ref, ew_ref, wvecs, nk):
    def body(k8, _):
        k = 8 * k8
        win = ew_ref[pl.ds(k, 16)]
        for e in range(8):
            s = jnp.full((16,), -win[e])
            for j in range(NG):
                sl = pl.ds(j * 16, 16)
                g = 1.0 / (1.0 + jnp.exp(s * wvecs[j]))
                rows_ref[k + e, sl] = rows_ref[k + e, sl] * g
        return 0
    lax.fori_loop(0, nk // 8, body, 0)


def _sc_body(h_hbm, col_hbm, row_hbm, ew_hbm, wedge_hbm, out_hbm,
             colA, rowA, ewA, colB, rowB, ewB, rows_v, rows_v2,
             colt, rowt, ewt, rowst, wedge_v, agg_sh,
             gsem, gsem2, isA, isB):
    sid = lax.axis_index("s")
    cid = lax.axis_index("c")
    wid = sid * NC + cid
    ebase = wid * EPT

    pltpu.sync_copy(wedge_hbm, wedge_v)

    def zbody(i, _):
        z = jnp.zeros((16,), jnp.float32)
        for j in range(NG):
            rows_v[i, pl.ds(j * 16, 16)] = z
        return 0
    lax.fori_loop(0, CH, zbody, 0)
    for z in range(ROWS_PT // CH):
        pltpu.sync_copy(rows_v, agg_sh.at[pl.ds(sid * ROWS_PT + z * CH, CH)])
    plsc.subcore_barrier()

    wvecs = [wedge_v[pl.ds(j * 16, 16)] for j in range(NG)]

    def ifetch(c, colv, rowv, ewv, isem):
        base = ebase + c * CH
        pltpu.async_copy(col_hbm.at[pl.ds(base, CH)], colv, isem)
        pltpu.async_copy(row_hbm.at[pl.ds(base, CH)], rowv, isem)
        pltpu.async_copy(ew_hbm.at[pl.ds(base, CH)], ewv.at[pl.ds(0, CH)],
                         isem)

    def iwait(c, colv, rowv, ewv, isem):
        base = ebase + c * CH
        pltpu.make_async_copy(col_hbm.at[pl.ds(base, CH)], colv, isem).wait()
        pltpu.make_async_copy(row_hbm.at[pl.ds(base, CH)], rowv, isem).wait()
        pltpu.make_async_copy(ew_hbm.at[pl.ds(base, CH)],
                              ewv.at[pl.ds(0, CH)], isem).wait()

    ifetch(0, colA, rowA, ewA, isA)
    ifetch(1, colB, rowB, ewB, isB)

    def pair(i, _):
        c0 = 2 * i
        iwait(c0, colA, rowA, ewA, isA)
        ga = pltpu.async_copy(h_hbm.at[colA], rows_v, gsem)
        iwait(c0 + 1, colB, rowB, ewB, isB)
        gb = pltpu.async_copy(h_hbm.at[colB], rows_v2, gsem2)
        ga.wait()
        _gate_rows(rows_v, ewA, wvecs, CH)
        pltpu.sync_copy(rows_v, agg_sh.at[rowA], add=True)
        @pl.when(c0 + 2 < NFULL)
        def _():
            ifetch(c0 + 2, colA, rowA, ewA, isA)
        gb.wait()
        _gate_rows(rows_v2, ewB, wvecs, CH)
        pltpu.sync_copy(rows_v2, agg_sh.at[rowB], add=True)
        @pl.when(c0 + 3 < NFULL)
        def _():
            ifetch(c0 + 3, colB, rowB, ewB, isB)
        return 0
    lax.fori_loop(0, NFULL // 2, pair, 0)

    tbase = ebase + NFULL * CH
    pltpu.sync_copy(col_hbm.at[pl.ds(tbase, TAIL)], colt)
    pltpu.sync_copy(row_hbm.at[pl.ds(tbase, TAIL)], rowt)
    pltpu.sync_copy(ew_hbm.at[pl.ds(tbase, TAIL)], ewt.at[pl.ds(0, TAIL)])
    pltpu.async_copy(h_hbm.at[colt], rowst, gsem).wait()
    _gate_rows(rowst, ewt, wvecs, TAIL)
    pltpu.sync_copy(rowst, agg_sh.at[rowt], add=True)

    plsc.subcore_barrier()
    pltpu.sync_copy(agg_sh.at[pl.ds(sid * ROWS_PT, ROWS_PT)],
                    out_hbm.at[cid, pl.ds(sid * ROWS_PT, ROWS_PT)])


@jax.jit
def _sc_aggregate(h_neigh, col, row, ew, w_edge):
    mesh = plsc.VectorSubcoreMesh(core_axis_name="c", subcore_axis_name="s")
    f = pl.kernel(
        _sc_body,
        out_type=jax.ShapeDtypeStruct((NC, NP, D), jnp.float32),
        mesh=mesh,
        scratch_types=[
            pltpu.VMEM((CH,), jnp.int32),
            pltpu.VMEM((CH,), jnp.int32),
            pltpu.VMEM((CH + 16,), jnp.float32),
            pltpu.VMEM((CH,), jnp.int32),
            pltpu.VMEM((CH,), jnp.int32),
            pltpu.VMEM((CH + 16,), jnp.float32),
            pltpu.VMEM((CH, D), jnp.float32),
            pltpu.VMEM((CH, D), jnp.float32),
            pltpu.VMEM((TAIL,), jnp.int32),
            pltpu.VMEM((TAIL,), jnp.int32),
            pltpu.VMEM((TAIL + 16,), jnp.float32),
            pltpu.VMEM((TAIL, D), jnp.float32),
            pltpu.VMEM((D,), jnp.float32),
            pltpu.VMEM_SHARED((NP, D), jnp.float32),
            pltpu.SemaphoreType.DMA,
            pltpu.SemaphoreType.DMA,
            pltpu.SemaphoreType.DMA,
            pltpu.SemaphoreType.DMA,
        ],
    )
    return f(h_neigh, col, row, ew, w_edge)


def _mm_body(x_ref, wt_ref, b_ref, o_ref):
    o_ref[...] = (
        jnp.dot(x_ref[...], wt_ref[...], preferred_element_type=jnp.float32)
        + b_ref[...]
    )


@jax.jit
def _mm(x, wt, b):
    bm = 400
    return pl.pallas_call(
        _mm_body,
        grid=(N // bm,),
        in_specs=[
            pl.BlockSpec((bm, D), lambda i: (i, 0)),
            pl.BlockSpec((D, D), lambda i: (0, 0)),
            pl.BlockSpec((1, D), lambda i: (0, 0)),
        ],
        out_specs=pl.BlockSpec((bm, D), lambda i: (i, 0)),
        out_shape=jax.ShapeDtypeStruct((N, D), jnp.float32),
    )(x, wt, b)


def _final_body(x_ref, wt_ref, b_ref, a_ref, g_ref, be_ref, o_ref):
    h = (
        jnp.dot(x_ref[...], wt_ref[...], preferred_element_type=jnp.float32)
        + b_ref[...]
        + a_ref[0]
        + a_ref[1]
    )
    mean = jnp.mean(h, axis=-1, keepdims=True)
    cent = h - mean
    var = jnp.mean(cent * cent, axis=-1, keepdims=True)
    y = cent * lax.rsqrt(var + 1e-5) * g_ref[...] + be_ref[...]
    o_ref[...] = jnp.where(y >= 0, y, 0.2 * y)


@jax.jit
def _final(x, wt, b, agg, gamma, beta):
    bm = 400
    return pl.pallas_call(
        _final_body,
        grid=(N // bm,),
        in_specs=[
            pl.BlockSpec((bm, D), lambda i: (i, 0)),
            pl.BlockSpec((D, D), lambda i: (0, 0)),
            pl.BlockSpec((1, D), lambda i: (0, 0)),
            pl.BlockSpec((NC, bm, D), lambda i: (0, i, 0)),
            pl.BlockSpec((1, D), lambda i: (0, 0)),
            pl.BlockSpec((1, D), lambda i: (0, 0)),
        ],
        out_specs=pl.BlockSpec((bm, D), lambda i: (i, 0)),
        out_shape=jax.ShapeDtypeStruct((N, D), jnp.float32),
    )(x, wt, b, agg, gamma, beta)


def kernel(x, edge_index, edge_weight, W_self, b_self, W_neigh, b_neigh,
           w_edge, ln_gamma, ln_beta):
    row = edge_index[0].astype(jnp.int32)
    col = edge_index[1].astype(jnp.int32)
    h_neigh = _mm(x, W_neigh.T, b_neigh.reshape(1, D))
    agg = _sc_aggregate(h_neigh, col, row, edge_weight, w_edge)
    return _final(x, W_self.T, b_self.reshape(1, D), agg,
                  ln_gamma.reshape(1, D), ln_beta.reshape(1, D))


# speedup vs baseline: 1.0658x; 1.0658x over previous
import functools
import jax
import jax.numpy as jnp
from jax import lax
from jax.experimental import pallas as pl
from jax.experimental.pallas import tpu as pltpu
from jax.experimental.pallas import tpu_sc as plsc

N = 10000
E = 320000
D = 128
NG = D // 16

NC = 2
NS = 16
NW = NC * NS
EPT = E // NW
CH = 128
NFULL = EPT // CH
TAIL = EPT - NFULL * CH
NP = 10240
ROWS_PT = NP // NS


def _gate_rows(rows_ref, ew_ref, wvecs, nk):
    def body(k8, _):
        k = 8 * k8
        win = ew_ref[pl.ds(k, 16)]
        for e in range(8):
            s = jnp.full((16,), -win[e])
            for j in range(NG):
                sl = pl.ds(j * 16, 16)
                g = 1.0 / (1.0 + jnp.exp(s * wvecs[j]))
                rows_ref[k + e, sl] = rows_ref[k + e, sl] * g
        return 0
    lax.fori_loop(0, nk // 8, body, 0)


def _sc_body(h_hbm, col_hbm, row_hbm, ew_hbm, wedge_hbm, out_hbm,
             colA, rowA, ewA, colB, rowB, ewB, rows_v, rows_v2,
             colt, rowt, ewt, rowst, wedge_v, agg_sh,
             gsem, gsem2, isA, isB):
    sid = lax.axis_index("s")
    cid = lax.axis_index("c")
    wid = sid * NC + cid
    ebase = wid * EPT

    pltpu.sync_copy(wedge_hbm, wedge_v)

    def zbody(i, _):
        z = jnp.zeros((16,), jnp.float32)
        for j in range(NG):
            rows_v[i, pl.ds(j * 16, 16)] = z
        return 0
    lax.fori_loop(0, CH, zbody, 0)
    for z in range(ROWS_PT // CH):
        pltpu.sync_copy(rows_v, agg_sh.at[pl.ds(sid * ROWS_PT + z * CH, CH)])
    plsc.subcore_barrier()

    wvecs = [wedge_v[pl.ds(j * 16, 16)] for j in range(NG)]

    def ifetch(c, colv, rowv, ewv, isem):
        base = ebase + c * CH
        pltpu.async_copy(col_hbm.at[pl.ds(base, CH)], colv, isem)
        pltpu.async_copy(row_hbm.at[pl.ds(base, CH)], rowv, isem)
        pltpu.async_copy(ew_hbm.at[pl.ds(base, CH)], ewv.at[pl.ds(0, CH)],
                         isem)

    def iwait(c, colv, rowv, ewv, isem):
        base = ebase + c * CH
        pltpu.make_async_copy(col_hbm.at[pl.ds(base, CH)], colv, isem).wait()
        pltpu.make_async_copy(row_hbm.at[pl.ds(base, CH)], rowv, isem).wait()
        pltpu.make_async_copy(ew_hbm.at[pl.ds(base, CH)],
                              ewv.at[pl.ds(0, CH)], isem).wait()

    ifetch(0, colA, rowA, ewA, isA)
    ifetch(1, colB, rowB, ewB, isB)

    def pair(i, _):
        c0 = 2 * i
        iwait(c0, colA, rowA, ewA, isA)
        ga = pltpu.async_copy(h_hbm.at[colA], rows_v, gsem)
        iwait(c0 + 1, colB, rowB, ewB, isB)
        gb = pltpu.async_copy(h_hbm.at[colB], rows_v2, gsem2)
        ga.wait()
        _gate_rows(rows_v, ewA, wvecs, CH)
        pltpu.sync_copy(rows_v, agg_sh.at[rowA], add=True)
        @pl.when(c0 + 2 < NFULL)
        def _():
            ifetch(c0 + 2, colA, rowA, ewA, isA)
        gb.wait()
        _gate_rows(rows_v2, ewB, wvecs, CH)
        pltpu.sync_copy(rows_v2, agg_sh.at[rowB], add=True)
        @pl.when(c0 + 3 < NFULL)
        def _():
            ifetch(c0 + 3, colB, rowB, ewB, isB)
        return 0
    lax.fori_loop(0, NFULL // 2, pair, 0)

    tbase = ebase + NFULL * CH
    pltpu.sync_copy(col_hbm.at[pl.ds(tbase, TAIL)], colt)
    pltpu.sync_copy(row_hbm.at[pl.ds(tbase, TAIL)], rowt)
    pltpu.sync_copy(ew_hbm.at[pl.ds(tbase, TAIL)], ewt.at[pl.ds(0, TAIL)])
    pltpu.async_copy(h_hbm.at[colt], rowst, gsem).wait()
    _gate_rows(rowst, ewt, wvecs, TAIL)
    pltpu.sync_copy(rowst, agg_sh.at[rowt], add=True)

    plsc.subcore_barrier()
    pltpu.sync_copy(agg_sh.at[pl.ds(sid * ROWS_PT, ROWS_PT)],
                    out_hbm.at[cid, pl.ds(sid * ROWS_PT, ROWS_PT)])


@jax.jit
def _sc_aggregate(h_neigh, col, row, ew, w_edge):
    mesh = plsc.VectorSubcoreMesh(core_axis_name="c", subcore_axis_name="s")
    f = pl.kernel(
        _sc_body,
        out_type=jax.ShapeDtypeStruct((NC, NP, D), jnp.float32),
        mesh=mesh,
        scratch_types=[
            pltpu.VMEM((CH,), jnp.int32),
            pltpu.VMEM((CH,), jnp.int32),
            pltpu.VMEM((CH + 16,), jnp.float32),
            pltpu.VMEM((CH,), jnp.int32),
            pltpu.VMEM((CH,), jnp.int32),
            pltpu.VMEM((CH + 16,), jnp.float32),
            pltpu.VMEM((CH, D), jnp.float32),
            pltpu.VMEM((CH, D), jnp.float32),
            pltpu.VMEM((TAIL,), jnp.int32),
            pltpu.VMEM((TAIL,), jnp.int32),
            pltpu.VMEM((TAIL + 16,), jnp.float32),
            pltpu.VMEM((TAIL, D), jnp.float32),
            pltpu.VMEM((D,), jnp.float32),
            pltpu.VMEM_SHARED((NP, D), jnp.float32),
            pltpu.SemaphoreType.DMA,
            pltpu.SemaphoreType.DMA,
            pltpu.SemaphoreType.DMA,
            pltpu.SemaphoreType.DMA,
        ],
    )
    return f(h_neigh, col, row, ew, w_edge)


def _mm_body(x_ref, wt_ref, b_ref, o_ref):
    o_ref[...] = (
        jnp.dot(x_ref[...], wt_ref[...], preferred_element_type=jnp.float32)
        + b_ref[...]
    )


@jax.jit
def _mm(x, wt, b):
    bm = 2000
    return pl.pallas_call(
        _mm_body,
        grid=(N // bm,),
        in_specs=[
            pl.BlockSpec((bm, D), lambda i: (i, 0)),
            pl.BlockSpec((D, D), lambda i: (0, 0)),
            pl.BlockSpec((1, D), lambda i: (0, 0)),
        ],
        out_specs=pl.BlockSpec((bm, D), lambda i: (i, 0)),
        out_shape=jax.ShapeDtypeStruct((N, D), jnp.float32),
    )(x, wt, b)


def _final_body(x_ref, wt_ref, b_ref, a_ref, g_ref, be_ref, o_ref):
    h = (
        jnp.dot(x_ref[...], wt_ref[...], preferred_element_type=jnp.float32)
        + b_ref[...]
        + a_ref[0]
        + a_ref[1]
    )
    mean = jnp.mean(h, axis=-1, keepdims=True)
    cent = h - mean
    var = jnp.mean(cent * cent, axis=-1, keepdims=True)
    y = cent * lax.rsqrt(var + 1e-5) * g_ref[...] + be_ref[...]
    o_ref[...] = jnp.where(y >= 0, y, 0.2 * y)


@jax.jit
def _final(x, wt, b, agg, gamma, beta):
    bm = 2000
    return pl.pallas_call(
        _final_body,
        grid=(N // bm,),
        in_specs=[
            pl.BlockSpec((bm, D), lambda i: (i, 0)),
            pl.BlockSpec((D, D), lambda i: (0, 0)),
            pl.BlockSpec((1, D), lambda i: (0, 0)),
            pl.BlockSpec((NC, bm, D), lambda i: (0, i, 0)),
            pl.BlockSpec((1, D), lambda i: (0, 0)),
            pl.BlockSpec((1, D), lambda i: (0, 0)),
        ],
        out_specs=pl.BlockSpec((bm, D), lambda i: (i, 0)),
        out_shape=jax.ShapeDtypeStruct((N, D), jnp.float32),
    )(x, wt, b, agg, gamma, beta)


def kernel(x, edge_index, edge_weight, W_self, b_self, W_neigh, b_neigh,
           w_edge, ln_gamma, ln_beta):
    row = edge_index[0].astype(jnp.int32)
    col = edge_index[1].astype(jnp.int32)
    h_neigh = _mm(x, W_neigh.T, b_neigh.reshape(1, D))
    agg = _sc_aggregate(h_neigh, col, row, edge_weight, w_edge)
    return _final(x, W_self.T, b_self.reshape(1, D), agg,
                  ln_gamma.reshape(1, D), ln_beta.reshape(1, D))


# speedup vs baseline: 1.0803x; 1.0136x over previous
import functools
import jax
import jax.numpy as jnp
from jax import lax
from jax.experimental import pallas as pl
from jax.experimental.pallas import tpu as pltpu
from jax.experimental.pallas import tpu_sc as plsc

N = 10000
E = 320000
D = 128
NG = D // 16

NC = 2
NS = 16
NW = NC * NS
EPT = E // NW
CH = 128
NFULL = EPT // CH
TAIL = EPT - NFULL * CH
NP = 10240
ROWS_PT = NP // NS


def _gate_rows(rows_ref, ew_ref, wvecs, nk):
    def body(k8, _):
        k = 8 * k8
        win = ew_ref[pl.ds(k, 16)]
        for e in range(8):
            s = jnp.full((16,), -win[e])
            for j in range(NG):
                sl = pl.ds(j * 16, 16)
                g = 1.0 / (1.0 + jnp.exp(s * wvecs[j]))
                rows_ref[k + e, sl] = rows_ref[k + e, sl] * g
        return 0
    lax.fori_loop(0, nk // 8, body, 0)


def _sc_body(h_hbm, col_hbm, row_hbm, ew_hbm, wedge_hbm, out_hbm,
             colA, rowA, ewA, colB, rowB, ewB, rows_v, rows_v2,
             colt, rowt, ewt, rowst, wedge_v, agg_sh,
             gsem, gsem2, isA, isB):
    sid = lax.axis_index("s")
    cid = lax.axis_index("c")
    wid = sid * NC + cid
    ebase = wid * EPT

    pltpu.sync_copy(wedge_hbm, wedge_v)

    def zbody(i, _):
        z = jnp.zeros((16,), jnp.float32)
        for j in range(NG):
            rows_v[i, pl.ds(j * 16, 16)] = z
        return 0
    lax.fori_loop(0, CH, zbody, 0)
    for z in range(ROWS_PT // CH):
        pltpu.sync_copy(rows_v, agg_sh.at[pl.ds(sid * ROWS_PT + z * CH, CH)])
    plsc.subcore_barrier()

    wvecs = [wedge_v[pl.ds(j * 16, 16)] for j in range(NG)]

    def ifetch(c, colv, rowv, ewv, isem):
        base = ebase + c * CH
        pltpu.async_copy(col_hbm.at[pl.ds(base, CH)], colv, isem)
        pltpu.async_copy(row_hbm.at[pl.ds(base, CH)], rowv, isem)
        pltpu.async_copy(ew_hbm.at[pl.ds(base, CH)], ewv.at[pl.ds(0, CH)],
                         isem)

    def iwait(c, colv, rowv, ewv, isem):
        base = ebase + c * CH
        pltpu.make_async_copy(col_hbm.at[pl.ds(base, CH)], colv, isem).wait()
        pltpu.make_async_copy(row_hbm.at[pl.ds(base, CH)], rowv, isem).wait()
        pltpu.make_async_copy(ew_hbm.at[pl.ds(base, CH)],
                              ewv.at[pl.ds(0, CH)], isem).wait()

    ifetch(0, colA, rowA, ewA, isA)
    ifetch(1, colB, rowB, ewB, isB)

    def pair(i, _):
        c0 = 2 * i
        iwait(c0, colA, rowA, ewA, isA)
        ga = pltpu.async_copy(h_hbm.at[colA], rows_v, gsem)
        iwait(c0 + 1, colB, rowB, ewB, isB)
        gb = pltpu.async_copy(h_hbm.at[colB], rows_v2, gsem2)
        ga.wait()
        _gate_rows(rows_v, ewA, wvecs, CH)
        pltpu.sync_copy(rows_v, agg_sh.at[rowA], add=True)
        @pl.when(c0 + 2 < NFULL)
        def _():
            ifetch(c0 + 2, colA, rowA, ewA, isA)
        gb.wait()
        _gate_rows(rows_v2, ewB, wvecs, CH)
        pltpu.sync_copy(rows_v2, agg_sh.at[rowB], add=True)
        @pl.when(c0 + 3 < NFULL)
        def _():
            ifetch(c0 + 3, colB, rowB, ewB, isB)
        return 0
    lax.fori_loop(0, NFULL // 2, pair, 0)

    tbase = ebase + NFULL * CH
    pltpu.sync_copy(col_hbm.at[pl.ds(tbase, TAIL)], colt)
    pltpu.sync_copy(row_hbm.at[pl.ds(tbase, TAIL)], rowt)
    pltpu.sync_copy(ew_hbm.at[pl.ds(tbase, TAIL)], ewt.at[pl.ds(0, TAIL)])
    pltpu.async_copy(h_hbm.at[colt], rowst, gsem).wait()
    _gate_rows(rowst, ewt, wvecs, TAIL)
    pltpu.sync_copy(rowst, agg_sh.at[rowt], add=True)

    plsc.subcore_barrier()
    pltpu.sync_copy(agg_sh.at[pl.ds(sid * ROWS_PT, ROWS_PT)],
                    out_hbm.at[cid, pl.ds(sid * ROWS_PT, ROWS_PT)])


@jax.jit
def _sc_aggregate(h_neigh, col, row, ew, w_edge):
    mesh = plsc.VectorSubcoreMesh(core_axis_name="c", subcore_axis_name="s")
    f = pl.kernel(
        _sc_body,
        out_type=jax.ShapeDtypeStruct((NC, NP, D), jnp.float32),
        mesh=mesh,
        scratch_types=[
            pltpu.VMEM((CH,), jnp.int32),
            pltpu.VMEM((CH,), jnp.int32),
            pltpu.VMEM((CH + 16,), jnp.float32),
            pltpu.VMEM((CH,), jnp.int32),
            pltpu.VMEM((CH,), jnp.int32),
            pltpu.VMEM((CH + 16,), jnp.float32),
            pltpu.VMEM((CH, D), jnp.float32),
            pltpu.VMEM((CH, D), jnp.float32),
            pltpu.VMEM((TAIL,), jnp.int32),
            pltpu.VMEM((TAIL,), jnp.int32),
            pltpu.VMEM((TAIL + 16,), jnp.float32),
            pltpu.VMEM((TAIL, D), jnp.float32),
            pltpu.VMEM((D,), jnp.float32),
            pltpu.VMEM_SHARED((NP, D), jnp.float32),
            pltpu.SemaphoreType.DMA,
            pltpu.SemaphoreType.DMA,
            pltpu.SemaphoreType.DMA,
            pltpu.SemaphoreType.DMA,
        ],
    )
    return f(h_neigh, col, row, ew, w_edge)


def _mm_body(x_ref, wt_ref, b_ref, o_ref):
    o_ref[...] = (
        jnp.dot(x_ref[...], wt_ref[...], preferred_element_type=jnp.float32)
        + b_ref[...]
    )


@jax.jit
def _mm(x, wt, b):
    bm = 5000
    return pl.pallas_call(
        _mm_body,
        grid=(N // bm,),
        in_specs=[
            pl.BlockSpec((bm, D), lambda i: (i, 0)),
            pl.BlockSpec((D, D), lambda i: (0, 0)),
            pl.BlockSpec((1, D), lambda i: (0, 0)),
        ],
        out_specs=pl.BlockSpec((bm, D), lambda i: (i, 0)),
        out_shape=jax.ShapeDtypeStruct((N, D), jnp.float32),
    )(x, wt, b)


def _final_body(x_ref, wt_ref, b_ref, a_ref, g_ref, be_ref, o_ref):
    h = (
        jnp.dot(x_ref[...], wt_ref[...], preferred_element_type=jnp.float32)
        + b_ref[...]
        + a_ref[0]
        + a_ref[1]
    )
    mean = jnp.mean(h, axis=-1, keepdims=True)
    cent = h - mean
    var = jnp.mean(cent * cent, axis=-1, keepdims=True)
    y = cent * lax.rsqrt(var + 1e-5) * g_ref[...] + be_ref[...]
    o_ref[...] = jnp.where(y >= 0, y, 0.2 * y)


@jax.jit
def _final(x, wt, b, agg, gamma, beta):
    bm = 5000
    return pl.pallas_call(
        _final_body,
        grid=(N // bm,),
        in_specs=[
            pl.BlockSpec((bm, D), lambda i: (i, 0)),
            pl.BlockSpec((D, D), lambda i: (0, 0)),
            pl.BlockSpec((1, D), lambda i: (0, 0)),
            pl.BlockSpec((NC, bm, D), lambda i: (0, i, 0)),
            pl.BlockSpec((1, D), lambda i: (0, 0)),
            pl.BlockSpec((1, D), lambda i: (0, 0)),
        ],
        out_specs=pl.BlockSpec((bm, D), lambda i: (i, 0)),
        out_shape=jax.ShapeDtypeStruct((N, D), jnp.float32),
    )(x, wt, b, agg, gamma, beta)


def kernel(x, edge_index, edge_weight, W_self, b_self, W_neigh, b_neigh,
           w_edge, ln_gamma, ln_beta):
    row = edge_index[0].astype(jnp.int32)
    col = edge_index[1].astype(jnp.int32)
    h_neigh = _mm(x, W_neigh.T, b_neigh.reshape(1, D))
    agg = _sc_aggregate(h_neigh, col, row, edge_weight, w_edge)
    return _final(x, W_self.T, b_self.reshape(1, D), agg,
                  ln_gamma.reshape(1, D), ln_beta.reshape(1, D))
